# Initial kernel scaffold; baseline (speedup 1.0000x reference)
import jax
import jax.numpy as jnp
from jax.experimental import pallas as pl


def kernel(x, conv0_w, conv0_b, conv1_w, conv1_b, conv2_w, conv2_b, fc0_w, fc0_b, fc1_w, fc1_b, fc2_w, fc2_b):
    raise NotImplementedError("write your pallas kernel here")


# speedup vs baseline: 18.1655x; 18.1655x over previous
import jax
import jax.numpy as jnp
from jax.experimental import pallas as pl
from jax.experimental.pallas import tpu as pltpu

_VMEM_LIMIT = 100 * 1024 * 1024


def _fused_vgg_kernel(x_ref, w0_ref, b0_ref, w1_ref, b1_ref, w2_ref, b2_ref,
                      f0w_ref, f0b_ref, f1w_ref, f1b_ref, f2w_ref, f2b_ref,
                      o_ref):
    B = x_ref.shape[0]
    f32 = jnp.float32
    bf16 = jnp.bfloat16

    h = jnp.dot(x_ref[...], w0_ref[...], preferred_element_type=f32)
    h = jnp.maximum(h + b0_ref[...], 0.0)
    h = h.reshape(B, 8, 4, 2, 128)
    h = jnp.maximum(h[:, :, :, 0, :], h[:, :, :, 1, :])
    h = h.reshape(B, 4, 2, 4, 128)
    h = jnp.maximum(h[:, :, 0, :, :], h[:, :, 1, :, :])
    h = h.astype(bf16)

    hp = jnp.pad(h, ((0, 0), (1, 1), (1, 1), (0, 0)))
    acc = jnp.zeros((B * 16, 256), f32)
    for t in range(9):
        dy, dx = divmod(t, 3)
        win = hp[:, dy:dy + 4, dx:dx + 4, :].reshape(B * 16, 128)
        acc = acc + jnp.dot(win, w1_ref[t], preferred_element_type=f32)
    h = jnp.maximum(acc + b1_ref[...], 0.0)
    h = h.reshape(B, 4, 2, 2, 256)
    h = jnp.maximum(h[:, :, :, 0, :], h[:, :, :, 1, :])
    h = h.reshape(B, 2, 2, 2, 256)
    h = jnp.maximum(h[:, :, 0, :, :], h[:, :, 1, :, :])
    h = h.astype(bf16).reshape(B, 1024)

    h = jnp.dot(h, w2_ref[...], preferred_element_type=f32)
    h = jnp.maximum(h + b2_ref[...], 0.0)
    h = h.reshape(B, 4, 512)
    h = jnp.max(h, axis=1)
    h = h.astype(bf16)

    h = jnp.dot(h, f0w_ref[...], preferred_element_type=f32)
    h = jnp.maximum(h + f0b_ref[...], 0.0).astype(bf16)
    h = jnp.dot(h, f1w_ref[...], preferred_element_type=f32)
    h = jnp.maximum(h + f1b_ref[...], 0.0).astype(bf16)
    h = jnp.dot(h, f2w_ref[...], preferred_element_type=f32)
    o_ref[...] = h + f2b_ref[...]


def _toeplitz_conv0(w9):
    w = w9.astype(jnp.float32)
    parts = []
    for t in range(9):
        dy, dx = divmod(t, 3)
        ey = jnp.eye(8, k=dy - 1, dtype=jnp.float32)
        ex = jnp.eye(8, k=dx - 1, dtype=jnp.float32)
        parts.append(jnp.einsum('oi,pj,cn->cijopn', ey, ex, w[t]))
    big = sum(parts)
    return big.reshape(192, 64 * 128).astype(jnp.bfloat16)


def _toeplitz_conv2(w9):
    w = w9.astype(jnp.float32)
    parts = []
    for t in range(9):
        dy, dx = divmod(t, 3)
        ey = jnp.eye(2, k=dy - 1, dtype=jnp.float32)
        ex = jnp.eye(2, k=dx - 1, dtype=jnp.float32)
        parts.append(jnp.einsum('oi,pj,cn->ijcopn', ey, ex, w[t]))
    big = sum(parts)
    return big.reshape(1024, 4 * 512).astype(jnp.bfloat16)


def kernel(x, conv0_w, conv0_b, conv1_w, conv1_b, conv2_w, conv2_b,
           fc0_w, fc0_b, fc1_w, fc1_b, fc2_w, fc2_b):
    n = x.shape[0]
    B = 256 if n % 256 == 0 else (128 if n % 128 == 0 else n)

    x2 = x.reshape(n, 192).astype(jnp.bfloat16)

    w0 = _toeplitz_conv0(conv0_w)
    b0 = jnp.tile(conv0_b, 64).reshape(1, 8192)
    w2 = _toeplitz_conv2(conv2_w)
    b2 = jnp.tile(conv2_b, 4).reshape(1, 2048)

    class_num = fc2_w.shape[1]
    npad = 128
    f2w = jnp.pad(fc2_w, ((0, 0), (0, npad - class_num)))
    f2b = jnp.pad(fc2_b, (0, npad - class_num)).reshape(1, npad)

    out = pl.pallas_call(
        _fused_vgg_kernel,
        grid=(n // B,),
        out_shape=jax.ShapeDtypeStruct((n, npad), jnp.float32),
        in_specs=[
            pl.BlockSpec((B, 192), lambda i: (i, 0)),
            pl.BlockSpec((192, 8192), lambda i: (0, 0)),
            pl.BlockSpec((1, 8192), lambda i: (0, 0)),
            pl.BlockSpec((9, 128, 256), lambda i: (0, 0, 0)),
            pl.BlockSpec((1, 256), lambda i: (0, 0)),
            pl.BlockSpec((1024, 2048), lambda i: (0, 0)),
            pl.BlockSpec((1, 2048), lambda i: (0, 0)),
            pl.BlockSpec((512, 512), lambda i: (0, 0)),
            pl.BlockSpec((1, 512), lambda i: (0, 0)),
            pl.BlockSpec((512, 512), lambda i: (0, 0)),
            pl.BlockSpec((1, 512), lambda i: (0, 0)),
            pl.BlockSpec((512, npad), lambda i: (0, 0)),
            pl.BlockSpec((1, npad), lambda i: (0, 0)),
        ],
        out_specs=pl.BlockSpec((B, npad), lambda i: (i, 0)),
        compiler_params=pltpu.CompilerParams(
            dimension_semantics=("parallel",),
            vmem_limit_bytes=_VMEM_LIMIT,
        ),
    )(x2, w0, b0, conv1_w, conv1_b.reshape(1, 256), w2, b2,
      fc0_w, fc0_b.reshape(1, 512), fc1_w, fc1_b.reshape(1, 512), f2w, f2b)
    return out[:, :class_num]


# speedup vs baseline: 62.5526x; 3.4435x over previous
import jax
import jax.numpy as jnp
from jax.experimental import pallas as pl
from jax.experimental.pallas import tpu as pltpu

_VMEM_LIMIT = 100 * 1024 * 1024


def _fused_vgg_kernel(x_ref, w0_ref, b0_ref, w1a_ref, w1b_ref, b1_ref,
                      w2_ref, b2_ref, f0w_ref, f0b_ref, f1w_ref, f1b_ref,
                      f2w_ref, f2b_ref, o_ref):
    B = x_ref.shape[0]
    f32 = jnp.float32
    bf16 = jnp.bfloat16

    h = jnp.dot(x_ref[...], w0_ref[...], preferred_element_type=f32)
    m = jnp.maximum(jnp.maximum(h[:, 0:2048], h[:, 2048:4096]),
                    jnp.maximum(h[:, 4096:6144], h[:, 6144:8192]))
    a1 = jnp.maximum(m + b0_ref[...], 0.0).astype(bf16)

    g0 = jnp.dot(a1[:, 0:512], w1a_ref[:, 0:2048], preferred_element_type=f32)
    g1 = jnp.dot(a1[:, 512:1024], w1b_ref[:, 0:3072], preferred_element_type=f32)
    g2 = jnp.dot(a1[:, 1024:1536], w1b_ref[:, 3072:6144], preferred_element_type=f32)
    g3 = jnp.dot(a1[:, 1536:2048], w1a_ref[:, 2048:4096], preferred_element_type=f32)
    oy0 = g0[:, 0:1024] + g1[:, 0:1024]
    oy1 = g0[:, 1024:2048] + g1[:, 1024:2048] + g2[:, 0:1024]
    oy2 = g1[:, 2048:3072] + g2[:, 1024:2048] + g3[:, 0:1024]
    oy3 = g2[:, 2048:3072] + g3[:, 1024:2048]
    r0 = jnp.maximum(jnp.maximum(oy0[:, 0:512], oy0[:, 512:1024]),
                     jnp.maximum(oy1[:, 0:512], oy1[:, 512:1024]))
    r1 = jnp.maximum(jnp.maximum(oy2[:, 0:512], oy2[:, 512:1024]),
                     jnp.maximum(oy3[:, 0:512], oy3[:, 512:1024]))
    r = jnp.concatenate([r0, r1], axis=1)
    a2 = jnp.maximum(r + b1_ref[...], 0.0).astype(bf16)

    g = jnp.dot(a2, w2_ref[...], preferred_element_type=f32)
    m = jnp.maximum(jnp.maximum(g[:, 0:512], g[:, 512:1024]),
                    jnp.maximum(g[:, 1024:1536], g[:, 1536:2048]))
    h = jnp.maximum(m + b2_ref[...], 0.0).astype(bf16)

    h = jnp.dot(h, f0w_ref[...], preferred_element_type=f32)
    h = jnp.maximum(h + f0b_ref[...], 0.0).astype(bf16)
    h = jnp.dot(h, f1w_ref[...], preferred_element_type=f32)
    h = jnp.maximum(h + f1b_ref[...], 0.0).astype(bf16)
    h = jnp.dot(h, f2w_ref[...], preferred_element_type=f32)
    o_ref[...] = h + f2b_ref[...]


def _toeplitz_conv0(w9):
    w = w9.astype(jnp.float32)
    parts = []
    for t in range(9):
        dy, dx = divmod(t, 3)
        ey = jnp.eye(8, k=dy - 1, dtype=jnp.float32)
        ex = jnp.eye(8, k=dx - 1, dtype=jnp.float32)
        parts.append(jnp.einsum('oi,pj,cn->cijopn', ey, ex, w[t]))
    big = sum(parts)
    big = big.reshape(3, 8, 8, 4, 2, 4, 2, 128)
    big = big.transpose(0, 1, 2, 4, 6, 3, 5, 7)
    return big.reshape(192, 8192).astype(jnp.bfloat16)


def _conv1_row_mats(w9):
    w = w9.astype(jnp.float32)
    bdy = []
    for dy in range(3):
        acc = 0
        for dx in range(3):
            ex = jnp.eye(4, k=dx - 1, dtype=jnp.float32)
            acc = acc + jnp.einsum('ox,cn->xcon', ex, w[dy * 3 + dx])
        acc = acc.reshape(4, 128, 2, 2, 256)
        acc = acc.transpose(0, 1, 3, 2, 4).reshape(512, 1024)
        bdy.append(acc)
    b0, b1, b2 = bdy
    w1_0 = jnp.concatenate([b1, b0], axis=1)
    w1_1 = jnp.concatenate([b2, b1, b0], axis=1)
    w1_2 = jnp.concatenate([b2, b1, b0], axis=1)
    w1_3 = jnp.concatenate([b2, b1], axis=1)
    w1a = jnp.concatenate([w1_0, w1_3], axis=1).astype(jnp.bfloat16)
    w1b = jnp.concatenate([w1_1, w1_2], axis=1).astype(jnp.bfloat16)
    return w1a, w1b


def _toeplitz_conv2(w9):
    w = w9.astype(jnp.float32)
    parts = []
    for t in range(9):
        dy, dx = divmod(t, 3)
        ey = jnp.eye(2, k=dy - 1, dtype=jnp.float32)
        ex = jnp.eye(2, k=dx - 1, dtype=jnp.float32)
        parts.append(jnp.einsum('oi,pj,cn->ijcopn', ey, ex, w[t]))
    big = sum(parts)
    return big.reshape(1024, 2048).astype(jnp.bfloat16)


def kernel(x, conv0_w, conv0_b, conv1_w, conv1_b, conv2_w, conv2_b,
           fc0_w, fc0_b, fc1_w, fc1_b, fc2_w, fc2_b):
    n = x.shape[0]
    B = 256 if n % 256 == 0 else (128 if n % 128 == 0 else n)

    x2 = x.reshape(n, 192).astype(jnp.bfloat16)

    w0 = _toeplitz_conv0(conv0_w)
    b0 = jnp.tile(conv0_b, 16).reshape(1, 2048)
    w1a, w1b = _conv1_row_mats(conv1_w)
    b1 = jnp.tile(conv1_b, 4).reshape(1, 1024)
    w2 = _toeplitz_conv2(conv2_w)
    b2 = conv2_b.reshape(1, 512)

    class_num = fc2_w.shape[1]
    npad = 128
    f2w = jnp.pad(fc2_w, ((0, 0), (0, npad - class_num)))
    f2b = jnp.pad(fc2_b, (0, npad - class_num)).reshape(1, npad)

    out = pl.pallas_call(
        _fused_vgg_kernel,
        grid=(n // B,),
        out_shape=jax.ShapeDtypeStruct((n, npad), jnp.float32),
        in_specs=[
            pl.BlockSpec((B, 192), lambda i: (i, 0)),
            pl.BlockSpec((192, 8192), lambda i: (0, 0)),
            pl.BlockSpec((1, 2048), lambda i: (0, 0)),
            pl.BlockSpec((512, 4096), lambda i: (0, 0)),
            pl.BlockSpec((512, 6144), lambda i: (0, 0)),
            pl.BlockSpec((1, 1024), lambda i: (0, 0)),
            pl.BlockSpec((1024, 2048), lambda i: (0, 0)),
            pl.BlockSpec((1, 512), lambda i: (0, 0)),
            pl.BlockSpec((512, 512), lambda i: (0, 0)),
            pl.BlockSpec((1, 512), lambda i: (0, 0)),
            pl.BlockSpec((512, 512), lambda i: (0, 0)),
            pl.BlockSpec((1, 512), lambda i: (0, 0)),
            pl.BlockSpec((512, npad), lambda i: (0, 0)),
            pl.BlockSpec((1, npad), lambda i: (0, 0)),
        ],
        out_specs=pl.BlockSpec((B, npad), lambda i: (i, 0)),
        compiler_params=pltpu.CompilerParams(
            dimension_semantics=("parallel",),
            vmem_limit_bytes=_VMEM_LIMIT,
        ),
    )(x2, w0, b0, w1a, w1b, b1, w2, b2,
      fc0_w, fc0_b.reshape(1, 512), fc1_w, fc1_b.reshape(1, 512), f2w, f2b)
    return out[:, :class_num]


# speedup vs baseline: 65.8882x; 1.0533x over previous
import jax
import jax.numpy as jnp
from jax.experimental import pallas as pl
from jax.experimental.pallas import tpu as pltpu

_VMEM_LIMIT = 100 * 1024 * 1024


def _fused_vgg_kernel(x_ref, w0_ref, b0_ref, w1a_ref, w1b_ref, b1_ref,
                      w2_ref, b2_ref, f0w_ref, f0b_ref, f1w_ref, f1b_ref,
                      f2w_ref, f2b_ref, o_ref):
    B = x_ref.shape[0]
    f32 = jnp.float32
    bf16 = jnp.bfloat16

    h = jnp.dot(x_ref[...], w0_ref[...], preferred_element_type=f32)
    m = jnp.maximum(jnp.maximum(h[:, 0:2048], h[:, 2048:4096]),
                    jnp.maximum(h[:, 4096:6144], h[:, 6144:8192]))
    a1 = jnp.maximum(m + b0_ref[...], 0.0).astype(bf16)

    g0 = jnp.dot(a1[:, 0:512], w1a_ref[:, 0:2048], preferred_element_type=f32)
    g1 = jnp.dot(a1[:, 512:1024], w1b_ref[:, 0:3072], preferred_element_type=f32)
    g2 = jnp.dot(a1[:, 1024:1536], w1b_ref[:, 3072:6144], preferred_element_type=f32)
    g3 = jnp.dot(a1[:, 1536:2048], w1a_ref[:, 2048:4096], preferred_element_type=f32)
    oy0 = g0[:, 0:1024] + g1[:, 0:1024]
    oy1 = g0[:, 1024:2048] + g1[:, 1024:2048] + g2[:, 0:1024]
    oy2 = g1[:, 2048:3072] + g2[:, 1024:2048] + g3[:, 0:1024]
    oy3 = g2[:, 2048:3072] + g3[:, 1024:2048]
    r0 = jnp.maximum(jnp.maximum(oy0[:, 0:512], oy0[:, 512:1024]),
                     jnp.maximum(oy1[:, 0:512], oy1[:, 512:1024]))
    r1 = jnp.maximum(jnp.maximum(oy2[:, 0:512], oy2[:, 512:1024]),
                     jnp.maximum(oy3[:, 0:512], oy3[:, 512:1024]))
    r = jnp.concatenate([r0, r1], axis=1)
    a2 = jnp.maximum(r + b1_ref[...], 0.0).astype(bf16)

    g = jnp.dot(a2, w2_ref[...], preferred_element_type=f32)
    m = jnp.maximum(jnp.maximum(g[:, 0:512], g[:, 512:1024]),
                    jnp.maximum(g[:, 1024:1536], g[:, 1536:2048]))
    h = jnp.maximum(m + b2_ref[...], 0.0).astype(bf16)

    h = jnp.dot(h, f0w_ref[...], preferred_element_type=f32)
    h = jnp.maximum(h + f0b_ref[...], 0.0).astype(bf16)
    h = jnp.dot(h, f1w_ref[...], preferred_element_type=f32)
    h = jnp.maximum(h + f1b_ref[...], 0.0).astype(bf16)
    h = jnp.dot(h, f2w_ref[...], preferred_element_type=f32)
    o_ref[...] = h + f2b_ref[...]


def _toeplitz_conv0(w9):
    w = w9.astype(jnp.float32)
    parts = []
    for t in range(9):
        dy, dx = divmod(t, 3)
        ey = jnp.eye(8, k=dy - 1, dtype=jnp.float32)
        ex = jnp.eye(8, k=dx - 1, dtype=jnp.float32)
        parts.append(jnp.einsum('oi,pj,cn->cijopn', ey, ex, w[t]))
    big = sum(parts)
    big = big.reshape(3, 8, 8, 4, 2, 4, 2, 128)
    big = big.transpose(0, 1, 2, 4, 6, 3, 5, 7)
    return big.reshape(192, 8192).astype(jnp.bfloat16)


def _conv1_row_mats(w9):
    w = w9.astype(jnp.float32)
    bdy = []
    for dy in range(3):
        acc = 0
        for dx in range(3):
            ex = jnp.eye(4, k=dx - 1, dtype=jnp.float32)
            acc = acc + jnp.einsum('ox,cn->xcon', ex, w[dy * 3 + dx])
        acc = acc.reshape(4, 128, 2, 2, 256)
        acc = acc.transpose(0, 1, 3, 2, 4).reshape(512, 1024)
        bdy.append(acc)
    b0, b1, b2 = bdy
    w1_0 = jnp.concatenate([b1, b0], axis=1)
    w1_1 = jnp.concatenate([b2, b1, b0], axis=1)
    w1_2 = jnp.concatenate([b2, b1, b0], axis=1)
    w1_3 = jnp.concatenate([b2, b1], axis=1)
    w1a = jnp.concatenate([w1_0, w1_3], axis=1).astype(jnp.bfloat16)
    w1b = jnp.concatenate([w1_1, w1_2], axis=1).astype(jnp.bfloat16)
    return w1a, w1b


def _toeplitz_conv2(w9):
    w = w9.astype(jnp.float32)
    parts = []
    for t in range(9):
        dy, dx = divmod(t, 3)
        ey = jnp.eye(2, k=dy - 1, dtype=jnp.float32)
        ex = jnp.eye(2, k=dx - 1, dtype=jnp.float32)
        parts.append(jnp.einsum('oi,pj,cn->ijcopn', ey, ex, w[t]))
    big = sum(parts)
    return big.reshape(1024, 2048).astype(jnp.bfloat16)


def kernel(x, conv0_w, conv0_b, conv1_w, conv1_b, conv2_w, conv2_b,
           fc0_w, fc0_b, fc1_w, fc1_b, fc2_w, fc2_b):
    n = x.shape[0]
    B = 512 if n % 512 == 0 else (128 if n % 128 == 0 else n)

    x2 = x.reshape(n, 192).astype(jnp.bfloat16)

    w0 = _toeplitz_conv0(conv0_w)
    b0 = jnp.tile(conv0_b, 16).reshape(1, 2048)
    w1a, w1b = _conv1_row_mats(conv1_w)
    b1 = jnp.tile(conv1_b, 4).reshape(1, 1024)
    w2 = _toeplitz_conv2(conv2_w)
    b2 = conv2_b.reshape(1, 512)

    class_num = fc2_w.shape[1]
    npad = 128
    f2w = jnp.pad(fc2_w, ((0, 0), (0, npad - class_num)))
    f2b = jnp.pad(fc2_b, (0, npad - class_num)).reshape(1, npad)

    out = pl.pallas_call(
        _fused_vgg_kernel,
        grid=(n // B,),
        out_shape=jax.ShapeDtypeStruct((n, npad), jnp.float32),
        in_specs=[
            pl.BlockSpec((B, 192), lambda i: (i, 0)),
            pl.BlockSpec((192, 8192), lambda i: (0, 0)),
            pl.BlockSpec((1, 2048), lambda i: (0, 0)),
            pl.BlockSpec((512, 4096), lambda i: (0, 0)),
            pl.BlockSpec((512, 6144), lambda i: (0, 0)),
            pl.BlockSpec((1, 1024), lambda i: (0, 0)),
            pl.BlockSpec((1024, 2048), lambda i: (0, 0)),
            pl.BlockSpec((1, 512), lambda i: (0, 0)),
            pl.BlockSpec((512, 512), lambda i: (0, 0)),
            pl.BlockSpec((1, 512), lambda i: (0, 0)),
            pl.BlockSpec((512, 512), lambda i: (0, 0)),
            pl.BlockSpec((1, 512), lambda i: (0, 0)),
            pl.BlockSpec((512, npad), lambda i: (0, 0)),
            pl.BlockSpec((1, npad), lambda i: (0, 0)),
        ],
        out_specs=pl.BlockSpec((B, npad), lambda i: (i, 0)),
        compiler_params=pltpu.CompilerParams(
            dimension_semantics=("parallel",),
            vmem_limit_bytes=_VMEM_LIMIT,
        ),
    )(x2, w0, b0, w1a, w1b, b1, w2, b2,
      fc0_w, fc0_b.reshape(1, 512), fc1_w, fc1_b.reshape(1, 512), f2w, f2b)
    return out[:, :class_num]


# speedup vs baseline: 66.6075x; 1.0109x over previous
import jax
import jax.numpy as jnp
from jax.experimental import pallas as pl
from jax.experimental.pallas import tpu as pltpu

_VMEM_LIMIT = 100 * 1024 * 1024


def _fused_vgg_kernel(x_ref, w0_ref, b0_ref, w1a_ref, w1b_ref, b1_ref,
                      w2_ref, b2_ref, f0w_ref, f0b_ref, f1w_ref, f1b_ref,
                      f2w_ref, f2b_ref, o_ref):
    B = x_ref.shape[0]
    f32 = jnp.float32
    bf16 = jnp.bfloat16

    h = jnp.dot(x_ref[...], w0_ref[...], preferred_element_type=f32)
    m = jnp.maximum(jnp.maximum(h[:, 0:2048], h[:, 2048:4096]),
                    jnp.maximum(h[:, 4096:6144], h[:, 6144:8192]))
    a1 = jnp.maximum(m + b0_ref[...], 0.0).astype(bf16)

    g0 = jnp.dot(a1[:, 0:512], w1a_ref[:, 0:2048], preferred_element_type=f32)
    g1 = jnp.dot(a1[:, 512:1024], w1b_ref[:, 0:3072], preferred_element_type=f32)
    g2 = jnp.dot(a1[:, 1024:1536], w1b_ref[:, 3072:6144], preferred_element_type=f32)
    g3 = jnp.dot(a1[:, 1536:2048], w1a_ref[:, 2048:4096], preferred_element_type=f32)
    oy0 = g0[:, 0:1024] + g1[:, 0:1024]
    oy1 = g0[:, 1024:2048] + g1[:, 1024:2048] + g2[:, 0:1024]
    oy2 = g1[:, 2048:3072] + g2[:, 1024:2048] + g3[:, 0:1024]
    oy3 = g2[:, 2048:3072] + g3[:, 1024:2048]
    r0 = jnp.maximum(jnp.maximum(oy0[:, 0:512], oy0[:, 512:1024]),
                     jnp.maximum(oy1[:, 0:512], oy1[:, 512:1024]))
    r1 = jnp.maximum(jnp.maximum(oy2[:, 0:512], oy2[:, 512:1024]),
                     jnp.maximum(oy3[:, 0:512], oy3[:, 512:1024]))
    r = jnp.concatenate([r0, r1], axis=1)
    a2 = jnp.maximum(r + b1_ref[...], 0.0).astype(bf16)

    g = jnp.dot(a2, w2_ref[...], preferred_element_type=f32)
    m = jnp.maximum(jnp.maximum(g[:, 0:512], g[:, 512:1024]),
                    jnp.maximum(g[:, 1024:1536], g[:, 1536:2048]))
    h = jnp.maximum(m + b2_ref[...], 0.0).astype(bf16)

    h = jnp.dot(h, f0w_ref[...], preferred_element_type=f32)
    h = jnp.maximum(h + f0b_ref[...], 0.0).astype(bf16)
    h = jnp.dot(h, f1w_ref[...], preferred_element_type=f32)
    h = jnp.maximum(h + f1b_ref[...], 0.0).astype(bf16)
    h = jnp.dot(h, f2w_ref[...], preferred_element_type=f32)
    o_ref[...] = h + f2b_ref[...]


def _toeplitz_conv0(w9):
    w = w9.astype(jnp.float32)
    parts = []
    for t in range(9):
        dy, dx = divmod(t, 3)
        ey = jnp.eye(8, k=dy - 1, dtype=jnp.float32)
        ex = jnp.eye(8, k=dx - 1, dtype=jnp.float32)
        parts.append(jnp.einsum('oi,pj,cn->cijopn', ey, ex, w[t]))
    big = sum(parts)
    big = big.reshape(3, 8, 8, 4, 2, 4, 2, 128)
    big = big.transpose(0, 1, 2, 4, 6, 3, 5, 7)
    return big.reshape(192, 8192).astype(jnp.bfloat16)


def _conv1_row_mats(w9):
    w = w9.astype(jnp.float32)
    bdy = []
    for dy in range(3):
        acc = 0
        for dx in range(3):
            ex = jnp.eye(4, k=dx - 1, dtype=jnp.float32)
            acc = acc + jnp.einsum('ox,cn->xcon', ex, w[dy * 3 + dx])
        acc = acc.reshape(4, 128, 2, 2, 256)
        acc = acc.transpose(0, 1, 3, 2, 4).reshape(512, 1024)
        bdy.append(acc)
    b0, b1, b2 = bdy
    w1_0 = jnp.concatenate([b1, b0], axis=1)
    w1_1 = jnp.concatenate([b2, b1, b0], axis=1)
    w1_2 = jnp.concatenate([b2, b1, b0], axis=1)
    w1_3 = jnp.concatenate([b2, b1], axis=1)
    w1a = jnp.concatenate([w1_0, w1_3], axis=1).astype(jnp.bfloat16)
    w1b = jnp.concatenate([w1_1, w1_2], axis=1).astype(jnp.bfloat16)
    return w1a, w1b


def _toeplitz_conv2(w9):
    w = w9.astype(jnp.float32)
    parts = []
    for t in range(9):
        dy, dx = divmod(t, 3)
        ey = jnp.eye(2, k=dy - 1, dtype=jnp.float32)
        ex = jnp.eye(2, k=dx - 1, dtype=jnp.float32)
        parts.append(jnp.einsum('oi,pj,cn->ijcopn', ey, ex, w[t]))
    big = sum(parts)
    return big.reshape(1024, 2048).astype(jnp.bfloat16)


def kernel(x, conv0_w, conv0_b, conv1_w, conv1_b, conv2_w, conv2_b,
           fc0_w, fc0_b, fc1_w, fc1_b, fc2_w, fc2_b):
    n = x.shape[0]
    B = 1024 if n % 1024 == 0 else (128 if n % 128 == 0 else n)

    x2 = x.reshape(n, 192).astype(jnp.bfloat16)

    w0 = _toeplitz_conv0(conv0_w)
    b0 = jnp.tile(conv0_b, 16).reshape(1, 2048)
    w1a, w1b = _conv1_row_mats(conv1_w)
    b1 = jnp.tile(conv1_b, 4).reshape(1, 1024)
    w2 = _toeplitz_conv2(conv2_w)
    b2 = conv2_b.reshape(1, 512)

    class_num = fc2_w.shape[1]
    npad = 128
    f2w = jnp.pad(fc2_w, ((0, 0), (0, npad - class_num)))
    f2b = jnp.pad(fc2_b, (0, npad - class_num)).reshape(1, npad)

    out = pl.pallas_call(
        _fused_vgg_kernel,
        grid=(n // B,),
        out_shape=jax.ShapeDtypeStruct((n, npad), jnp.float32),
        in_specs=[
            pl.BlockSpec((B, 192), lambda i: (i, 0)),
            pl.BlockSpec((192, 8192), lambda i: (0, 0)),
            pl.BlockSpec((1, 2048), lambda i: (0, 0)),
            pl.BlockSpec((512, 4096), lambda i: (0, 0)),
            pl.BlockSpec((512, 6144), lambda i: (0, 0)),
            pl.BlockSpec((1, 1024), lambda i: (0, 0)),
            pl.BlockSpec((1024, 2048), lambda i: (0, 0)),
            pl.BlockSpec((1, 512), lambda i: (0, 0)),
            pl.BlockSpec((512, 512), lambda i: (0, 0)),
            pl.BlockSpec((1, 512), lambda i: (0, 0)),
            pl.BlockSpec((512, 512), lambda i: (0, 0)),
            pl.BlockSpec((1, 512), lambda i: (0, 0)),
            pl.BlockSpec((512, npad), lambda i: (0, 0)),
            pl.BlockSpec((1, npad), lambda i: (0, 0)),
        ],
        out_specs=pl.BlockSpec((B, npad), lambda i: (i, 0)),
        compiler_params=pltpu.CompilerParams(
            dimension_semantics=("parallel",),
            vmem_limit_bytes=_VMEM_LIMIT,
        ),
    )(x2, w0, b0, w1a, w1b, b1, w2, b2,
      fc0_w, fc0_b.reshape(1, 512), fc1_w, fc1_b.reshape(1, 512), f2w, f2b)
    return out[:, :class_num]
